# baseline (speedup 1.0000x reference)
import jax
import jax.numpy as jnp
from jax.experimental import pallas as pl

S = 2048
H = 768
HD = 4002
HOUT = 4000
D1, V1 = 192, 16000
D2, V2 = 48, 80000
V = HOUT + V1 + V2

RB = 128
CS = 3200

BW = 2048
NB = (V + BW - 1) // BW
R2 = 1024
N1B = (V1 + BW - 1) // BW
N2B = (V2 + BW - 1) // BW
SH1 = BW - (HOUT - BW)
SH2 = BW - (HOUT + V1 - 9 * BW)

NEG = -1e30


def _stats_kernel(x_ref, hw_ref, hb_ref, p1w_ref, p1b_ref, t1w_ref, t1b_ref,
                  p2w_ref, p2b_ref, t2w_ref, t2b_ref,
                  head_ref, proj1_ref, proj2_ref, stats_ref):
    x = x_ref[0].astype(jnp.bfloat16)
    zh = jnp.dot(x, hw_ref[:].astype(jnp.bfloat16),
                 preferred_element_type=jnp.float32)
    zh = zh + hb_ref[0, :][None, :]
    mh = jnp.max(zh, axis=-1, keepdims=True)
    eh = jnp.exp(zh - mh)
    inv_sh = 1.0 / jnp.sum(eh, axis=-1, keepdims=True)
    head_ref[:] = eh * inv_sh
    g1 = eh[:, HOUT:HOUT + 1] * inv_sh
    g2 = eh[:, HOUT + 1:HOUT + 2] * inv_sh

    p1 = jnp.dot(x, p1w_ref[:].astype(jnp.bfloat16),
                 preferred_element_type=jnp.float32)
    p1 = (p1 + p1b_ref[0, :][None, :]).astype(jnp.bfloat16)
    proj1_ref[:] = p1
    p2 = jnp.dot(x, p2w_ref[:].astype(jnp.bfloat16),
                 preferred_element_type=jnp.float32)
    p2 = (p2 + p2b_ref[0, :][None, :]).astype(jnp.bfloat16)
    proj2_ref[:] = p2

    def tail_stats(p, w_ref, b_ref, v):
        def body(i, carry):
            m, s = carry
            sl = pl.ds(i * CS, CS)
            z = jnp.dot(p, w_ref[:, sl].astype(jnp.bfloat16),
                        preferred_element_type=jnp.float32)
            z = z + b_ref[0, sl][None, :]
            mc = jnp.max(z, axis=-1, keepdims=True)
            mn = jnp.maximum(m, mc)
            s = s * jnp.exp(m - mn) + jnp.sum(jnp.exp(z - mn), axis=-1,
                                              keepdims=True)
            return mn, s
        m0 = jnp.full((p.shape[0], 1), NEG, dtype=jnp.float32)
        s0 = jnp.zeros((p.shape[0], 1), dtype=jnp.float32)
        return jax.lax.fori_loop(0, v // CS, body, (m0, s0))

    m1, s1 = tail_stats(p1, t1w_ref, t1b_ref, V1)
    m2, s2 = tail_stats(p2, t2w_ref, t2b_ref, V2)

    zeros = jnp.zeros_like(m1)
    stats_ref[:] = jnp.concatenate(
        [m1, g1 / s1, m2, g2 / s2, zeros, zeros, zeros, zeros], axis=1)


def _write_kernel(head_ref, proj1_ref, proj2_ref, stats_ref,
                  w1p_ref, w1c_ref, b1p_ref, b1c_ref,
                  w2p_ref, w2c_ref, b2p_ref, b2c_ref, out_ref):
    j = pl.program_id(1)

    def store(v):
        out_ref[...] = v[None, :, :]

    def t1_val():
        w = jnp.concatenate([w1p_ref[:, SH1:].astype(jnp.bfloat16),
                             w1c_ref[:, :SH1].astype(jnp.bfloat16)], axis=1)
        b = jnp.concatenate([b1p_ref[0, SH1:], b1c_ref[0, :SH1]])[None, :]
        z = jnp.dot(proj1_ref[:], w, preferred_element_type=jnp.float32) + b
        return jnp.exp(z - stats_ref[:, 0:1]) * stats_ref[:, 1:2]

    def t2_val():
        w = jnp.concatenate([w2p_ref[:, SH2:].astype(jnp.bfloat16),
                             w2c_ref[:, :SH2].astype(jnp.bfloat16)], axis=1)
        b = jnp.concatenate([b2p_ref[0, SH2:], b2c_ref[0, :SH2]])[None, :]
        z = jnp.dot(proj2_ref[:], w, preferred_element_type=jnp.float32) + b
        return jnp.exp(z - stats_ref[:, 2:3]) * stats_ref[:, 3:4]

    def cols():
        return (j * BW
                + jax.lax.broadcasted_iota(jnp.int32, (1, BW), 1))

    @pl.when(j == 0)
    def _():
        store(head_ref[:])

    @pl.when(j == 1)
    def _():
        store(jnp.where(cols() < HOUT, head_ref[:], t1_val()))

    @pl.when(jnp.logical_and(j >= 2, j <= 8))
    def _():
        store(t1_val())

    @pl.when(j == 9)
    def _():
        store(jnp.where(cols() < HOUT + V1, t1_val(), t2_val()))

    @pl.when(j >= 10)
    def _():
        store(t2_val())


def kernel(inp, head_w, head_b, t1_pw, t1_pb, t1_w, t1_b,
           t2_pw, t2_pb, t2_w, t2_b):
    hb = head_b.reshape(1, HD)
    p1b = t1_pb.reshape(1, D1)
    p2b = t2_pb.reshape(1, D2)
    t1b2 = t1_b.reshape(1, V1)
    t2b2 = t2_b.reshape(1, V2)

    full = lambda shape: pl.BlockSpec(shape, lambda i: (0,) * len(shape))
    head, proj1, proj2, stats = pl.pallas_call(
        _stats_kernel,
        grid=(S // RB,),
        in_specs=[
            pl.BlockSpec((1, RB, H), lambda i: (0, i, 0)),
            full((H, HD)), full((1, HD)),
            full((H, D1)), full((1, D1)), full((D1, V1)), full((1, V1)),
            full((H, D2)), full((1, D2)), full((D2, V2)), full((1, V2)),
        ],
        out_specs=[
            pl.BlockSpec((RB, HD), lambda i: (i, 0)),
            pl.BlockSpec((RB, D1), lambda i: (i, 0)),
            pl.BlockSpec((RB, D2), lambda i: (i, 0)),
            pl.BlockSpec((RB, 8), lambda i: (i, 0)),
        ],
        out_shape=[
            jax.ShapeDtypeStruct((S, HD), jnp.float32),
            jax.ShapeDtypeStruct((S, D1), jnp.bfloat16),
            jax.ShapeDtypeStruct((S, D2), jnp.bfloat16),
            jax.ShapeDtypeStruct((S, 8), jnp.float32),
        ],
    )(inp, head_w, hb, t1_pw, p1b, t1_w, t1b2, t2_pw, p2b, t2_w, t2b2)

    out = pl.pallas_call(
        _write_kernel,
        grid=(S // R2, NB),
        in_specs=[
            pl.BlockSpec((R2, BW), lambda i, j: (i, jnp.minimum(j, 1))),
            pl.BlockSpec((R2, D1), lambda i, j: (i, 0)),
            pl.BlockSpec((R2, D2), lambda i, j: (i, 0)),
            pl.BlockSpec((R2, 8), lambda i, j: (i, 0)),
            pl.BlockSpec((D1, BW), lambda i, j: (0, jnp.clip(j - 2, 0, N1B - 1))),
            pl.BlockSpec((D1, BW), lambda i, j: (0, jnp.clip(j - 1, 0, N1B - 1))),
            pl.BlockSpec((1, BW), lambda i, j: (0, jnp.clip(j - 2, 0, N1B - 1))),
            pl.BlockSpec((1, BW), lambda i, j: (0, jnp.clip(j - 1, 0, N1B - 1))),
            pl.BlockSpec((D2, BW), lambda i, j: (0, jnp.clip(j - 10, 0, N2B - 1))),
            pl.BlockSpec((D2, BW), lambda i, j: (0, jnp.clip(j - 9, 0, N2B - 1))),
            pl.BlockSpec((1, BW), lambda i, j: (0, jnp.clip(j - 10, 0, N2B - 1))),
            pl.BlockSpec((1, BW), lambda i, j: (0, jnp.clip(j - 9, 0, N2B - 1))),
        ],
        out_specs=pl.BlockSpec((1, R2, BW), lambda i, j: (0, i, j)),
        out_shape=jax.ShapeDtypeStruct((1, S, V), jnp.float32),
    )(head, proj1, proj2, stats, t1_w, t1_w, t1b2, t1b2, t2_w, t2_w, t2b2, t2b2)

    return out


# speedup vs baseline: 1.3705x; 1.3705x over previous
import jax
import jax.numpy as jnp
from jax.experimental import pallas as pl

S = 2048
H = 768
HD = 4002
HOUT = 4000
D1, V1 = 192, 16000
D2, V2 = 48, 80000
V = HOUT + V1 + V2

RB = 128
CS = 3200

BW = 2048
NB = (V + BW - 1) // BW
R2 = 512
N1B = (V1 + BW - 1) // BW
N2B = (V2 + BW - 1) // BW
SH1 = BW - (HOUT - BW)
SH2 = BW - (HOUT + V1 - 9 * BW)

NEG = -1e30


def _stats_kernel(x_ref, hw_ref, hb_ref, p1w_ref, p1b_ref, t1w_ref, t1b_ref,
                  p2w_ref, p2b_ref, t2w_ref, t2b_ref,
                  headt_ref, proj1t_ref, proj2t_ref, statst_ref):
    x = x_ref[0].astype(jnp.bfloat16)
    zh = jnp.dot(x, hw_ref[:].astype(jnp.bfloat16),
                 preferred_element_type=jnp.float32)
    zh = zh + hb_ref[0, :][None, :]
    mh = jnp.max(zh, axis=-1, keepdims=True)
    eh = jnp.exp(zh - mh)
    inv_sh = 1.0 / jnp.sum(eh, axis=-1, keepdims=True)
    headt_ref[:] = (eh * inv_sh).T
    g1 = eh[:, HOUT:HOUT + 1] * inv_sh
    g2 = eh[:, HOUT + 1:HOUT + 2] * inv_sh

    p1 = jnp.dot(x, p1w_ref[:].astype(jnp.bfloat16),
                 preferred_element_type=jnp.float32)
    p1 = (p1 + p1b_ref[0, :][None, :]).astype(jnp.bfloat16)
    proj1t_ref[:] = p1.T
    p2 = jnp.dot(x, p2w_ref[:].astype(jnp.bfloat16),
                 preferred_element_type=jnp.float32)
    p2 = (p2 + p2b_ref[0, :][None, :]).astype(jnp.bfloat16)
    proj2t_ref[:] = p2.T

    def tail_stats(p, w_ref, b_ref, v):
        def body(i, carry):
            m, s = carry
            sl = pl.ds(i * CS, CS)
            z = jnp.dot(p, w_ref[:, sl].astype(jnp.bfloat16),
                        preferred_element_type=jnp.float32)
            z = z + b_ref[0, sl][None, :]
            mc = jnp.max(z, axis=-1, keepdims=True)
            mn = jnp.maximum(m, mc)
            s = s * jnp.exp(m - mn) + jnp.sum(jnp.exp(z - mn), axis=-1,
                                              keepdims=True)
            return mn, s
        m0 = jnp.full((p.shape[0], 1), NEG, dtype=jnp.float32)
        s0 = jnp.zeros((p.shape[0], 1), dtype=jnp.float32)
        return jax.lax.fori_loop(0, v // CS, body, (m0, s0))

    m1, s1 = tail_stats(p1, t1w_ref, t1b_ref, V1)
    m2, s2 = tail_stats(p2, t2w_ref, t2b_ref, V2)

    zeros = jnp.zeros_like(m1)
    statst_ref[:] = jnp.concatenate(
        [m1, g1 / s1, m2, g2 / s2, zeros, zeros, zeros, zeros], axis=1).T


def _write_kernel(headt_ref, proj1t_ref, proj2t_ref, statst_ref,
                  w1p_ref, w1c_ref, b1p_ref, b1c_ref,
                  w2p_ref, w2c_ref, b2p_ref, b2c_ref, out_ref):
    j = pl.program_id(1)

    def store(v):
        out_ref[...] = v[None, :, :]

    def t1_val():
        w = jnp.concatenate([w1p_ref[SH1:, :].astype(jnp.bfloat16),
                             w1c_ref[:SH1, :].astype(jnp.bfloat16)], axis=0)
        b = jnp.concatenate([b1p_ref[SH1:, :], b1c_ref[:SH1, :]], axis=0)
        z = jnp.dot(w, proj1t_ref[:], preferred_element_type=jnp.float32) + b
        return jnp.exp(z - statst_ref[0:1, :]) * statst_ref[1:2, :]

    def t2_val():
        w = jnp.concatenate([w2p_ref[SH2:, :].astype(jnp.bfloat16),
                             w2c_ref[:SH2, :].astype(jnp.bfloat16)], axis=0)
        b = jnp.concatenate([b2p_ref[SH2:, :], b2c_ref[:SH2, :]], axis=0)
        z = jnp.dot(w, proj2t_ref[:], preferred_element_type=jnp.float32) + b
        return jnp.exp(z - statst_ref[2:3, :]) * statst_ref[3:4, :]

    def rows():
        return (j * BW
                + jax.lax.broadcasted_iota(jnp.int32, (BW, 1), 0))

    @pl.when(j == 0)
    def _():
        store(headt_ref[:])

    @pl.when(j == 1)
    def _():
        store(jnp.where(rows() < HOUT, headt_ref[:], t1_val()))

    @pl.when(jnp.logical_and(j >= 2, j <= 8))
    def _():
        store(t1_val())

    @pl.when(j == 9)
    def _():
        store(jnp.where(rows() < HOUT + V1, t1_val(), t2_val()))

    @pl.when(j >= 10)
    def _():
        store(t2_val())


def kernel(inp, head_w, head_b, t1_pw, t1_pb, t1_w, t1_b,
           t2_pw, t2_pb, t2_w, t2_b):
    hb = head_b.reshape(1, HD)
    p1b = t1_pb.reshape(1, D1)
    p2b = t2_pb.reshape(1, D2)
    t1b2 = t1_b.reshape(1, V1)
    t2b2 = t2_b.reshape(1, V2)

    full = lambda shape: pl.BlockSpec(shape, lambda i: (0,) * len(shape))
    headt, proj1t, proj2t, statst = pl.pallas_call(
        _stats_kernel,
        grid=(S // RB,),
        in_specs=[
            pl.BlockSpec((1, RB, H), lambda i: (0, i, 0)),
            full((H, HD)), full((1, HD)),
            full((H, D1)), full((1, D1)), full((D1, V1)), full((1, V1)),
            full((H, D2)), full((1, D2)), full((D2, V2)), full((1, V2)),
        ],
        out_specs=[
            pl.BlockSpec((HD, RB), lambda i: (0, i)),
            pl.BlockSpec((D1, RB), lambda i: (0, i)),
            pl.BlockSpec((D2, RB), lambda i: (0, i)),
            pl.BlockSpec((8, RB), lambda i: (0, i)),
        ],
        out_shape=[
            jax.ShapeDtypeStruct((HD, S), jnp.float32),
            jax.ShapeDtypeStruct((D1, S), jnp.bfloat16),
            jax.ShapeDtypeStruct((D2, S), jnp.bfloat16),
            jax.ShapeDtypeStruct((8, S), jnp.float32),
        ],
    )(inp, head_w, hb, t1_pw, p1b, t1_w, t1b2, t2_pw, p2b, t2_w, t2b2)

    t1wt = t1_w.T
    t2wt = t2_w.T
    t1bt = t1_b.reshape(V1, 1)
    t2bt = t2_b.reshape(V2, 1)

    out = pl.pallas_call(
        _write_kernel,
        grid=(S // R2, NB),
        in_specs=[
            pl.BlockSpec((BW, R2), lambda i, j: (jnp.minimum(j, 1), i)),
            pl.BlockSpec((D1, R2), lambda i, j: (0, i)),
            pl.BlockSpec((D2, R2), lambda i, j: (0, i)),
            pl.BlockSpec((8, R2), lambda i, j: (0, i)),
            pl.BlockSpec((BW, D1), lambda i, j: (jnp.clip(j - 2, 0, N1B - 1), 0)),
            pl.BlockSpec((BW, D1), lambda i, j: (jnp.clip(j - 1, 0, N1B - 1), 0)),
            pl.BlockSpec((BW, 1), lambda i, j: (jnp.clip(j - 2, 0, N1B - 1), 0)),
            pl.BlockSpec((BW, 1), lambda i, j: (jnp.clip(j - 1, 0, N1B - 1), 0)),
            pl.BlockSpec((BW, D2), lambda i, j: (jnp.clip(j - 10, 0, N2B - 1), 0)),
            pl.BlockSpec((BW, D2), lambda i, j: (jnp.clip(j - 9, 0, N2B - 1), 0)),
            pl.BlockSpec((BW, 1), lambda i, j: (jnp.clip(j - 10, 0, N2B - 1), 0)),
            pl.BlockSpec((BW, 1), lambda i, j: (jnp.clip(j - 9, 0, N2B - 1), 0)),
        ],
        out_specs=pl.BlockSpec((1, BW, R2), lambda i, j: (0, j, i)),
        out_shape=jax.ShapeDtypeStruct((1, V, S), jnp.float32),
    )(headt, proj1t, proj2t, statst,
      t1wt, t1wt, t1bt, t1bt, t2wt, t2wt, t2bt, t2bt)

    return jnp.swapaxes(out, 1, 2)


# speedup vs baseline: 1.6297x; 1.1891x over previous
import jax
import jax.numpy as jnp
from jax.experimental import pallas as pl

S = 2048
H = 768
HD = 4002
HOUT = 4000
D1, V1 = 192, 16000
D2, V2 = 48, 80000
V = HOUT + V1 + V2

RB = 128
CS = 3200

BW = 512
NB = (V + BW - 1) // BW
R2 = 2048
N1B = (V1 + BW - 1) // BW
N2B = (V2 + BW - 1) // BW
A1Q = HOUT // BW
A2Q = (HOUT + V1) // BW
NHB = (HD + BW - 1) // BW
SH1 = BW - HOUT % BW
SH2 = BW - (HOUT + V1) % BW

NEG = -1e30


def _stats_kernel(x_ref, hw_ref, hb_ref, p1w_ref, p1b_ref, t1w_ref, t1b_ref,
                  p2w_ref, p2b_ref, t2w_ref, t2b_ref,
                  headt_ref, proj1t_ref, proj2t_ref, statst_ref):
    x = x_ref[0].astype(jnp.bfloat16)
    zh = jnp.dot(x, hw_ref[:].astype(jnp.bfloat16),
                 preferred_element_type=jnp.float32)
    zh = zh + hb_ref[0, :][None, :]
    mh = jnp.max(zh, axis=-1, keepdims=True)
    eh = jnp.exp(zh - mh)
    inv_sh = 1.0 / jnp.sum(eh, axis=-1, keepdims=True)
    headt_ref[:] = (eh * inv_sh).T
    g1 = eh[:, HOUT:HOUT + 1] * inv_sh
    g2 = eh[:, HOUT + 1:HOUT + 2] * inv_sh

    p1 = jnp.dot(x, p1w_ref[:].astype(jnp.bfloat16),
                 preferred_element_type=jnp.float32)
    p1 = (p1 + p1b_ref[0, :][None, :]).astype(jnp.bfloat16)
    proj1t_ref[:] = p1.T
    p2 = jnp.dot(x, p2w_ref[:].astype(jnp.bfloat16),
                 preferred_element_type=jnp.float32)
    p2 = (p2 + p2b_ref[0, :][None, :]).astype(jnp.bfloat16)
    proj2t_ref[:] = p2.T

    def tail_stats(p, w_ref, b_ref, v):
        def body(i, carry):
            m, s = carry
            sl = pl.ds(i * CS, CS)
            z = jnp.dot(p, w_ref[:, sl].astype(jnp.bfloat16),
                        preferred_element_type=jnp.float32)
            z = z + b_ref[0, sl][None, :]
            mc = jnp.max(z, axis=-1, keepdims=True)
            mn = jnp.maximum(m, mc)
            s = s * jnp.exp(m - mn) + jnp.sum(jnp.exp(z - mn), axis=-1,
                                              keepdims=True)
            return mn, s
        m0 = jnp.full((p.shape[0], 1), NEG, dtype=jnp.float32)
        s0 = jnp.zeros((p.shape[0], 1), dtype=jnp.float32)
        return jax.lax.fori_loop(0, v // CS, body, (m0, s0))

    m1, s1 = tail_stats(p1, t1w_ref, t1b_ref, V1)
    m2, s2 = tail_stats(p2, t2w_ref, t2b_ref, V2)

    zeros = jnp.zeros_like(m1)
    statst_ref[:] = jnp.concatenate(
        [m1, g1 / s1, m2, g2 / s2, zeros, zeros, zeros, zeros], axis=1).T


def _write_kernel(headt_ref, proj1t_ref, proj2t_ref, statst_ref,
                  w1p_ref, w1c_ref, b1p_ref, b1c_ref,
                  w2p_ref, w2c_ref, b2p_ref, b2c_ref, out_ref):
    j = pl.program_id(1)

    def store(v):
        out_ref[...] = v[None, :, :]

    def t1_val():
        w = jnp.concatenate([w1p_ref[SH1:, :].astype(jnp.bfloat16),
                             w1c_ref[:SH1, :].astype(jnp.bfloat16)], axis=0)
        b = jnp.concatenate([b1p_ref[SH1:, :], b1c_ref[:SH1, :]], axis=0)
        z = jnp.dot(w, proj1t_ref[:], preferred_element_type=jnp.float32) + b
        return jnp.exp(z - statst_ref[0:1, :]) * statst_ref[1:2, :]

    def t2_val():
        w = jnp.concatenate([w2p_ref[SH2:, :].astype(jnp.bfloat16),
                             w2c_ref[:SH2, :].astype(jnp.bfloat16)], axis=0)
        b = jnp.concatenate([b2p_ref[SH2:, :], b2c_ref[:SH2, :]], axis=0)
        z = jnp.dot(w, proj2t_ref[:], preferred_element_type=jnp.float32) + b
        return jnp.exp(z - statst_ref[2:3, :]) * statst_ref[3:4, :]

    def rows():
        return (j * BW
                + jax.lax.broadcasted_iota(jnp.int32, (BW, 1), 0))

    @pl.when(j < A1Q)
    def _():
        store(headt_ref[:])

    @pl.when(j == A1Q)
    def _():
        store(jnp.where(rows() < HOUT, headt_ref[:], t1_val()))

    @pl.when(jnp.logical_and(j > A1Q, j < A2Q))
    def _():
        store(t1_val())

    @pl.when(j == A2Q)
    def _():
        store(jnp.where(rows() < HOUT + V1, t1_val(), t2_val()))

    @pl.when(j > A2Q)
    def _():
        store(t2_val())


def kernel(inp, head_w, head_b, t1_pw, t1_pb, t1_w, t1_b,
           t2_pw, t2_pb, t2_w, t2_b):
    hb = head_b.reshape(1, HD)
    p1b = t1_pb.reshape(1, D1)
    p2b = t2_pb.reshape(1, D2)
    t1b2 = t1_b.reshape(1, V1)
    t2b2 = t2_b.reshape(1, V2)

    full = lambda shape: pl.BlockSpec(shape, lambda i: (0,) * len(shape))
    headt, proj1t, proj2t, statst = pl.pallas_call(
        _stats_kernel,
        grid=(S // RB,),
        in_specs=[
            pl.BlockSpec((1, RB, H), lambda i: (0, i, 0)),
            full((H, HD)), full((1, HD)),
            full((H, D1)), full((1, D1)), full((D1, V1)), full((1, V1)),
            full((H, D2)), full((1, D2)), full((D2, V2)), full((1, V2)),
        ],
        out_specs=[
            pl.BlockSpec((HD, RB), lambda i: (0, i)),
            pl.BlockSpec((D1, RB), lambda i: (0, i)),
            pl.BlockSpec((D2, RB), lambda i: (0, i)),
            pl.BlockSpec((8, RB), lambda i: (0, i)),
        ],
        out_shape=[
            jax.ShapeDtypeStruct((HD, S), jnp.float32),
            jax.ShapeDtypeStruct((D1, S), jnp.bfloat16),
            jax.ShapeDtypeStruct((D2, S), jnp.bfloat16),
            jax.ShapeDtypeStruct((8, S), jnp.float32),
        ],
    )(inp, head_w, hb, t1_pw, p1b, t1_w, t1b2, t2_pw, p2b, t2_w, t2b2)

    t1wt = t1_w.T
    t2wt = t2_w.T
    t1bt = t1_b.reshape(V1, 1)
    t2bt = t2_b.reshape(V2, 1)

    out = pl.pallas_call(
        _write_kernel,
        grid=(S // R2, NB),
        in_specs=[
            pl.BlockSpec((BW, R2), lambda i, j: (jnp.minimum(j, NHB - 1), i)),
            pl.BlockSpec((D1, R2), lambda i, j: (0, i)),
            pl.BlockSpec((D2, R2), lambda i, j: (0, i)),
            pl.BlockSpec((8, R2), lambda i, j: (0, i)),
            pl.BlockSpec((BW, D1), lambda i, j: (jnp.clip(j - A1Q - 1, 0, N1B - 1), 0)),
            pl.BlockSpec((BW, D1), lambda i, j: (jnp.clip(j - A1Q, 0, N1B - 1), 0)),
            pl.BlockSpec((BW, 1), lambda i, j: (jnp.clip(j - A1Q - 1, 0, N1B - 1), 0)),
            pl.BlockSpec((BW, 1), lambda i, j: (jnp.clip(j - A1Q, 0, N1B - 1), 0)),
            pl.BlockSpec((BW, D2), lambda i, j: (jnp.clip(j - A2Q - 1, 0, N2B - 1), 0)),
            pl.BlockSpec((BW, D2), lambda i, j: (jnp.clip(j - A2Q, 0, N2B - 1), 0)),
            pl.BlockSpec((BW, 1), lambda i, j: (jnp.clip(j - A2Q - 1, 0, N2B - 1), 0)),
            pl.BlockSpec((BW, 1), lambda i, j: (jnp.clip(j - A2Q, 0, N2B - 1), 0)),
        ],
        out_specs=pl.BlockSpec((1, BW, R2), lambda i, j: (0, j, i)),
        out_shape=jax.ShapeDtypeStruct((1, V, S), jnp.float32),
    )(headt, proj1t, proj2t, statst,
      t1wt, t1wt, t1bt, t1bt, t2wt, t2wt, t2bt, t2bt)

    return jnp.swapaxes(out, 1, 2)
